# baseline (speedup 1.0000x reference)
import functools

import jax
import jax.numpy as jnp
from jax.experimental import pallas as pl
from jax.experimental.pallas import tpu as pltpu

_NSLOTS = 8
_M = 128
_NH = 2


def _bilstm_mean_relu_kernel(x_hbm, wia_ref, whh_ref, b_ref, out_ref,
                             gx_ref, xbuf, h_ref, c_ref, acc_ref, sem,
                             *, H, B_BLK):
    _, T, _ = x_hbm.shape
    _, _, G = gx_ref.shape
    GP = G // 4
    inv_T = 1.0 / T
    base = pl.program_id(0) * B_BLK
    n_pairs = T // 2
    SB = B_BLK // _NH
    n_jobs = _NH * n_pairs
    chunks = [(s, min(_M, SB - s)) for s in range(0, SB, _M)]

    def job_copies(j, slot):
        q, k = divmod(j, n_pairs)
        b0 = base + q * SB
        return (pltpu.make_async_copy(x_hbm.at[pl.ds(b0, SB), k, :],
                                      xbuf.at[slot, 0], sem.at[slot, 0]),
                pltpu.make_async_copy(x_hbm.at[pl.ds(b0, SB), T - 1 - k, :],
                                      xbuf.at[slot, 1], sem.at[slot, 1]))

    wia = wia_ref[...]
    bias = b_ref[...]
    whh = whh_ref[...]

    lane = jax.lax.broadcasted_iota(jnp.int32, (1, G), 1) % GP
    bwd_mask = jnp.logical_and(lane >= H, lane < 2 * H)

    h_ref[...] = jnp.zeros((B_BLK, GP), jnp.float32)
    c_ref[...] = jnp.zeros((B_BLK, GP), jnp.float32)
    acc_ref[...] = jnp.zeros((B_BLK, GP), jnp.float32)

    def step_chunk(gates, sl):
        ti = jnp.tanh(gates[:, 0 * GP:1 * GP])
        tf = jnp.tanh(gates[:, 1 * GP:2 * GP])
        g = jnp.tanh(gates[:, 2 * GP:3 * GP])
        to = jnp.tanh(gates[:, 3 * GP:4 * GP])
        c = c_ref[sl, :]
        c = 0.5 * (tf * c + c + ti * g + g)
        c_ref[sl, :] = c
        tc = jnp.tanh(c)
        hm = 0.5 * (to * tc + tc)
        h_ref[sl, :] = hm
        acc_ref[sl, :] += hm

    def proj_and_step(j, slot):
        q, k = divmod(j, n_pairs)
        for s, w in chunks:
            sl = pl.ds(q * SB + s, w)
            xs = pl.ds(s, w)
            p1 = jnp.dot(xbuf[slot, 0, xs, :], wia,
                         preferred_element_type=jnp.float32) + bias
            p2 = jnp.dot(xbuf[slot, 1, xs, :], wia,
                         preferred_element_type=jnp.float32) + bias
            gx_ref[n_pairs - 1 - k, sl, :] = jnp.where(bwd_mask, p1, p2)
            gates = jnp.where(bwd_mask, p2, p1) + jnp.dot(
                h_ref[sl, :], whh, preferred_element_type=jnp.float32)
            step_chunk(gates, sl)

    def tail_step(t, q):
        for s, w in chunks:
            sl = pl.ds(q * SB + s, w)
            gates = gx_ref[t - n_pairs, sl, :] + jnp.dot(
                h_ref[sl, :], whh, preferred_element_type=jnp.float32)
            step_chunk(gates, sl)

    for j in range(min(_NSLOTS - 1, n_jobs)):
        for cp in job_copies(j, j % _NSLOTS):
            cp.start()
    for j in range(n_jobs):
        jp = j + _NSLOTS - 1
        if jp < n_jobs:
            for cp in job_copies(jp, jp % _NSLOTS):
                cp.start()
        for cp in job_copies(j, j % _NSLOTS):
            cp.wait()
        proj_and_step(j, j % _NSLOTS)
        if j >= n_pairs:
            tail_step(n_pairs + (j % n_pairs), j // n_pairs - 1)

    def tail(t, carry):
        tail_step(t, _NH - 1)
        return carry

    jax.lax.fori_loop(n_pairs, T, tail, 0, unroll=8)

    out_ref[...] = jnp.maximum(acc_ref[...] * inv_T, 0.0)


def kernel(x, wia, whh, b):
    B, T, D = x.shape
    GP = whh.shape[0]
    G = wia.shape[1]
    H = GP // 2

    col = jnp.arange(G) // GP
    scale = jnp.where(col == 2, 1.0, 0.5).astype(jnp.float32)
    wia_s = wia * scale
    whh_s = whh * scale
    b_s = b * scale

    Bp = max(8, -(-B // 8) * 8)
    B_BLK = min(Bp, 1024)
    Bp = -(-Bp // B_BLK) * B_BLK
    if Bp != B:
        x = jnp.pad(x, ((0, Bp - B), (0, 0), (0, 0)))

    assert T % 2 == 0 and B_BLK % _NH == 0
    body = functools.partial(_bilstm_mean_relu_kernel, H=H, B_BLK=B_BLK)

    out = pl.pallas_call(
        body,
        out_shape=jax.ShapeDtypeStruct((Bp, GP), jnp.float32),
        grid=(Bp // B_BLK,),
        in_specs=[
            pl.BlockSpec(memory_space=pltpu.MemorySpace.HBM),
            pl.BlockSpec(wia_s.shape, lambda i: (0, 0)),
            pl.BlockSpec(whh_s.shape, lambda i: (0, 0)),
            pl.BlockSpec(b_s.shape, lambda i: (0, 0)),
        ],
        out_specs=pl.BlockSpec((B_BLK, GP), lambda i: (i, 0)),
        scratch_shapes=[
            pltpu.VMEM((T // 2, B_BLK, G), jnp.float32),
            pltpu.VMEM((_NSLOTS, 2, B_BLK // _NH, D), jnp.float32),
            pltpu.VMEM((B_BLK, GP), jnp.float32),
            pltpu.VMEM((B_BLK, GP), jnp.float32),
            pltpu.VMEM((B_BLK, GP), jnp.float32),
            pltpu.SemaphoreType.DMA((_NSLOTS, 2)),
        ],
        compiler_params=pltpu.CompilerParams(
            dimension_semantics=("parallel",),
            vmem_limit_bytes=56 * 1024 * 1024,
        ),
    )(x, wia_s, whh_s, b_s)

    return out[:B, :GP]


# speedup vs baseline: 1.0856x; 1.0856x over previous
import functools

import jax
import jax.numpy as jnp
from jax.experimental import pallas as pl
from jax.experimental.pallas import tpu as pltpu

_NSLOTS = 4
_M = 128
_NH = 1


def _bilstm_mean_relu_kernel(x_hbm, wia_ref, whh_ref, b_ref, out_ref,
                             gx_ref, xbuf, h_ref, c_ref, acc_ref, sem,
                             *, H, B_BLK):
    _, T, _ = x_hbm.shape
    _, _, G = gx_ref.shape
    GP = G // 4
    inv_T = 1.0 / T
    base = pl.program_id(0) * B_BLK
    n_pairs = T // 2
    SB = B_BLK // _NH
    n_jobs = _NH * n_pairs
    chunks = [(s, min(_M, SB - s)) for s in range(0, SB, _M)]

    def job_copies(j, slot):
        q, k = divmod(j, n_pairs)
        b0 = base + q * SB
        return (pltpu.make_async_copy(x_hbm.at[pl.ds(b0, SB), k, :],
                                      xbuf.at[slot, 0], sem.at[slot, 0]),
                pltpu.make_async_copy(x_hbm.at[pl.ds(b0, SB), T - 1 - k, :],
                                      xbuf.at[slot, 1], sem.at[slot, 1]))

    wia = wia_ref[...]
    bias = b_ref[...]
    whh = whh_ref[...]

    lane = jax.lax.broadcasted_iota(jnp.int32, (1, G), 1) % GP
    bwd_mask = jnp.logical_and(lane >= H, lane < 2 * H)

    h_ref[...] = jnp.zeros((B_BLK, GP), jnp.float32)
    c_ref[...] = jnp.zeros((B_BLK, GP), jnp.float32)
    acc_ref[...] = jnp.zeros((B_BLK, GP), jnp.float32)

    def step_chunk(gates, sl):
        ti = jnp.tanh(gates[:, 0 * GP:1 * GP])
        tf = jnp.tanh(gates[:, 1 * GP:2 * GP])
        g = jnp.tanh(gates[:, 2 * GP:3 * GP])
        to = jnp.tanh(gates[:, 3 * GP:4 * GP])
        c = c_ref[sl, :]
        c = 0.5 * (tf * c + c + ti * g + g)
        c_ref[sl, :] = c
        tc = jnp.tanh(c)
        hm = 0.5 * (to * tc + tc)
        h_ref[sl, :] = hm
        acc_ref[sl, :] += hm

    def proj_and_step(j, slot):
        q, k = divmod(j, n_pairs)
        for s, w in chunks:
            sl = pl.ds(q * SB + s, w)
            xs = pl.ds(s, w)
            p1 = jnp.dot(xbuf[slot, 0, xs, :], wia,
                         preferred_element_type=jnp.float32) + bias
            p2 = jnp.dot(xbuf[slot, 1, xs, :], wia,
                         preferred_element_type=jnp.float32) + bias
            gx_ref[n_pairs - 1 - k, sl, :] = jnp.where(bwd_mask, p1, p2)
            gates = jnp.where(bwd_mask, p2, p1) + jnp.dot(
                h_ref[sl, :], whh, preferred_element_type=jnp.float32)
            step_chunk(gates, sl)

    def tail_step(t, q):
        for s, w in chunks:
            sl = pl.ds(q * SB + s, w)
            gates = gx_ref[t - n_pairs, sl, :] + jnp.dot(
                h_ref[sl, :], whh, preferred_element_type=jnp.float32)
            step_chunk(gates, sl)

    for j in range(min(_NSLOTS - 1, n_jobs)):
        for cp in job_copies(j, j % _NSLOTS):
            cp.start()
    for j in range(n_jobs):
        jp = j + _NSLOTS - 1
        if jp < n_jobs:
            for cp in job_copies(jp, jp % _NSLOTS):
                cp.start()
        for cp in job_copies(j, j % _NSLOTS):
            cp.wait()
        proj_and_step(j, j % _NSLOTS)
        if j >= n_pairs:
            tail_step(n_pairs + (j % n_pairs), j // n_pairs - 1)

    def tail(t, carry):
        tail_step(t, _NH - 1)
        return carry

    jax.lax.fori_loop(n_pairs, T, tail, 0, unroll=8)

    out_ref[...] = jnp.maximum(acc_ref[...] * inv_T, 0.0)


def kernel(x, wia, whh, b):
    B, T, D = x.shape
    GP = whh.shape[0]
    G = wia.shape[1]
    H = GP // 2

    col = jnp.arange(G) // GP
    scale = jnp.where(col == 2, 1.0, 0.5).astype(jnp.float32)
    wia_s = wia * scale
    whh_s = whh * scale
    b_s = b * scale

    Bp = max(8, -(-B // 8) * 8)
    B_BLK = min(Bp, 1024)
    Bp = -(-Bp // B_BLK) * B_BLK
    if Bp != B:
        x = jnp.pad(x, ((0, Bp - B), (0, 0), (0, 0)))

    assert T % 2 == 0 and B_BLK % _NH == 0
    body = functools.partial(_bilstm_mean_relu_kernel, H=H, B_BLK=B_BLK)

    out = pl.pallas_call(
        body,
        out_shape=jax.ShapeDtypeStruct((Bp, GP), jnp.float32),
        grid=(Bp // B_BLK,),
        in_specs=[
            pl.BlockSpec(memory_space=pltpu.MemorySpace.HBM),
            pl.BlockSpec(wia_s.shape, lambda i: (0, 0)),
            pl.BlockSpec(whh_s.shape, lambda i: (0, 0)),
            pl.BlockSpec(b_s.shape, lambda i: (0, 0)),
        ],
        out_specs=pl.BlockSpec((B_BLK, GP), lambda i: (i, 0)),
        scratch_shapes=[
            pltpu.VMEM((T // 2, B_BLK, G), jnp.float32),
            pltpu.VMEM((_NSLOTS, 2, B_BLK // _NH, D), jnp.float32),
            pltpu.VMEM((B_BLK, GP), jnp.float32),
            pltpu.VMEM((B_BLK, GP), jnp.float32),
            pltpu.VMEM((B_BLK, GP), jnp.float32),
            pltpu.SemaphoreType.DMA((_NSLOTS, 2)),
        ],
        compiler_params=pltpu.CompilerParams(
            dimension_semantics=("parallel",),
            vmem_limit_bytes=56 * 1024 * 1024,
        ),
    )(x, wia_s, whh_s, b_s)

    return out[:B, :GP]
